# baseline (speedup 1.0000x reference)
import functools

import jax
import jax.numpy as jnp
from jax import lax
from jax.experimental import pallas as pl
from jax.experimental.pallas import tpu as pltpu
from jax.experimental.pallas import tpu_sc as plsc

N_USERS = 30000
N_ITEMS = 20000
N = N_USERS + N_ITEMS
D = 32
NNZ = 1600000
B = 4096
T_CONST = 2.0
GAMMA = 1.0

NPAD = 50048
NC, NS, L = 2, 16, 16
NW = NC * NS
EPW = NNZ // NW
SUB = 80
KSUB = 5
CHUNK = SUB * KSUB
NCH = EPW // CHUNK
ZROWS = 391


def _spmm_kernel(adj_row, adj_col, adj_val, emb):
    mesh = plsc.VectorSubcoreMesh(core_axis_name="c", subcore_axis_name="s")

    @functools.partial(
        pl.kernel,
        mesh=mesh,
        out_type=(
            jax.ShapeDtypeStruct((NPAD, D), jnp.float32),
            jax.ShapeDtypeStruct((NPAD, D), jnp.float32),
        ),
        scratch_types=[
            pltpu.VMEM((2, KSUB, SUB), jnp.int32),
            pltpu.VMEM((2, KSUB, SUB), jnp.int32),
            pltpu.VMEM((2, KSUB, SUB), jnp.float32),
            pltpu.VMEM((2, KSUB, SUB), jnp.int32),
            pltpu.VMEM((CHUNK, D), jnp.float32),
            pltpu.VMEM((CHUNK, D), jnp.float32),
            pltpu.VMEM_SHARED((NPAD, D), jnp.float32),
            pltpu.SemaphoreType.DMA,
            pltpu.SemaphoreType.DMA,
            pltpu.SemaphoreType.DMA,
            pltpu.SemaphoreType.DMA,
            pltpu.SemaphoreType.DMA,
            pltpu.SemaphoreType.DMA,
        ],
        compiler_params=pltpu.CompilerParams(use_tc_tiling_on_sc=False),
    )
    def k(rows_h, cols_h, vals_h, emb_h, out0, out1, colv, rowv, valv, rsc,
          gbuf0, gbuf1, accum, sl0, sl1, sg0, sg1, ss0, ss1):
        cid = lax.axis_index("c")
        sid = lax.axis_index("s")
        wid = cid * NS + sid
        gbufs = (gbuf0, gbuf1)
        sls = (sl0, sl1)
        sgs = (sg0, sg1)
        sss = (ss0, ss1)

        zeros16 = jnp.zeros((L,), jnp.float32)

        def zrow(i, _):
            gbuf0[i, pl.ds(0, L)] = zeros16
            gbuf0[i, pl.ds(L, L)] = zeros16
            return 0

        lax.fori_loop(0, ZROWS, zrow, 0)

        def zcopy(q, _):
            pltpu.sync_copy(gbuf0.at[pl.ds(0, ZROWS)],
                            accum.at[pl.ds(sid * (NPAD // NS) + q * ZROWS, ZROWS)])
            return 0

        lax.fori_loop(0, (NPAD // NS) // ZROWS, zcopy, 0)
        plsc.subcore_barrier()

        rbase = wid * (EPW // SUB)

        def fire_loads(c, b):
            r0 = rbase + c * KSUB
            pltpu.async_copy(cols_h.at[pl.ds(r0, KSUB)], colv.at[b], sls[b])
            pltpu.async_copy(rows_h.at[pl.ds(r0, KSUB)], rowv.at[b], sls[b])
            pltpu.async_copy(vals_h.at[pl.ds(r0, KSUB)], valv.at[b], sls[b])

        def drain_loads(b):
            pltpu.make_async_copy(cols_h.at[pl.ds(0, KSUB)], colv.at[b],
                                  sls[b]).wait()
            pltpu.make_async_copy(rows_h.at[pl.ds(0, KSUB)], rowv.at[b],
                                  sls[b]).wait()
            pltpu.make_async_copy(vals_h.at[pl.ds(0, KSUB)], valv.at[b],
                                  sls[b]).wait()

        def fire_gathers(b):
            for j in range(KSUB):
                pltpu.async_copy(emb_h.at[colv.at[b, j]],
                                 gbufs[b].at[pl.ds(j * SUB, SUB)], sgs[b])

        def drain_gathers(b):
            for j in range(KSUB):
                pltpu.make_async_copy(emb_h.at[pl.ds(0, SUB)],
                                      gbufs[b].at[pl.ds(j * SUB, SUB)],
                                      sgs[b]).wait()

        def scale(b):
            g_ref = gbufs[b]
            for j in range(KSUB):
                def grp(g, _, j=j):
                    v16 = valv[b, j, pl.ds(g * L, L)]
                    for u in range(L):
                        e = j * SUB + g * L + u
                        vb = v16.at[jnp.full((L,), u, jnp.int32)].get(
                            mode="promise_in_bounds")
                        a = g_ref[e, pl.ds(0, L)] * vb
                        bb = g_ref[e, pl.ds(L, L)] * vb
                        g_ref[e, pl.ds(0, L)] = a
                        g_ref[e, pl.ds(L, L)] = bb
                    return 0

                lax.fori_loop(0, SUB // L, grp, 0)

        def copy_scatter_idx(b):
            for j in range(KSUB):
                def cp(g, _, j=j):
                    rsc[b, j, pl.ds(g * L, L)] = rowv[b, j, pl.ds(g * L, L)]
                    return 0

                lax.fori_loop(0, SUB // L, cp, 0)

        def fire_scatter(b):
            for j in range(KSUB):
                pltpu.async_copy(gbufs[b].at[pl.ds(j * SUB, SUB)],
                                 accum.at[rsc.at[b, j]], sss[b], add=True)

        def drain_scatter(b):
            for j in range(KSUB):
                pltpu.make_async_copy(gbufs[b].at[pl.ds(j * SUB, SUB)],
                                      accum.at[pl.ds(0, SUB)], sss[b]).wait()

        def step(c, b, first, last):
            drain_gathers(b)
            drain_loads(1 - b)
            if not first:
                drain_scatter(1 - b)
            fire_gathers(1 - b)
            copy_scatter_idx(b)
            scale(b)
            fire_scatter(b)
            r_next = jnp.minimum(c + 2, NCH - 1)
            fire_loads(r_next, b)

        fire_loads(0, 0)
        drain_loads(0)
        fire_gathers(0)
        fire_loads(1, 1)
        step(0, 0, first=True, last=False)
        step(1, 1, first=False, last=False)

        def pair(p, _):
            for b in (0, 1):
                step(2 * p + 2 + b, b, first=False, last=False)
            return 0

        lax.fori_loop(0, (NCH - 3) // 2, pair, 0)

        drain_gathers(0)
        drain_scatter(1)
        copy_scatter_idx(0)
        scale(0)
        fire_scatter(0)
        drain_scatter(0)
        drain_loads(1)
        plsc.subcore_barrier()

        def rd(q, _):
            r0 = sid * (NPAD // NS) + q * ZROWS
            pltpu.sync_copy(accum.at[pl.ds(r0, ZROWS)], gbuf0.at[pl.ds(0, ZROWS)])

            @pl.when(cid == 0)
            def _():
                pltpu.sync_copy(gbuf0.at[pl.ds(0, ZROWS)], out0.at[pl.ds(r0, ZROWS)])

            @pl.when(cid == 1)
            def _():
                pltpu.sync_copy(gbuf0.at[pl.ds(0, ZROWS)], out1.at[pl.ds(r0, ZROWS)])
            return 0

        lax.fori_loop(0, (NPAD // NS) // ZROWS, rd, 0)

    nr = NNZ // SUB
    return k(adj_row.reshape(nr, SUB), adj_col.reshape(nr, SUB),
             adj_val.reshape(nr, SUB), emb)


def _sc_gather3(t0, t1, t2, idx, nrows):
    mesh = plsc.VectorSubcoreMesh(core_axis_name="c", subcore_axis_name="s")
    per_w = nrows // NW
    kq = per_w // 128

    @functools.partial(
        pl.kernel,
        mesh=mesh,
        out_type=jax.ShapeDtypeStruct((nrows, D), jnp.float32),
        scratch_types=[
            pltpu.VMEM((kq, 128), jnp.int32),
            pltpu.VMEM((per_w, D), jnp.float32),
            pltpu.VMEM((per_w, D), jnp.float32),
            pltpu.VMEM((per_w, D), jnp.float32),
            pltpu.SemaphoreType.DMA,
        ],
        compiler_params=pltpu.CompilerParams(use_tc_tiling_on_sc=False),
    )
    def k(t0_h, t1_h, t2_h, idx_h, out_h, idxv, b0, b1, b2, sem):
        wid = lax.axis_index("c") * NS + lax.axis_index("s")
        pltpu.sync_copy(idx_h.at[pl.ds(wid * kq, kq)], idxv)
        copies = []
        for tab, buf in ((t0_h, b0), (t1_h, b1), (t2_h, b2)):
            for q in range(kq):
                copies.append(
                    pltpu.async_copy(tab.at[idxv.at[q]],
                                     buf.at[pl.ds(q * 128, 128)], sem))
        for cpy in copies:
            cpy.wait()

        def addrow(r, _):
            a0 = b0[r, pl.ds(0, L)] + b1[r, pl.ds(0, L)] + b2[r, pl.ds(0, L)]
            a1 = b0[r, pl.ds(L, L)] + b1[r, pl.ds(L, L)] + b2[r, pl.ds(L, L)]
            b0[r, pl.ds(0, L)] = a0
            b0[r, pl.ds(L, L)] = a1
            return 0

        lax.fori_loop(0, per_w, addrow, 0)
        pltpu.sync_copy(b0, out_h.at[pl.ds(wid * per_w, per_w)])

    return k(t0, t1, t2, idx.reshape(nrows // 128, 128))


def _tc_add2(a, b):
    def body(a_ref, b_ref, o_ref):
        o_ref[...] = a_ref[...] + b_ref[...]

    blk = pl.BlockSpec((NPAD // 16, D), lambda i: (i, 0))
    return pl.pallas_call(
        body,
        grid=(16,),
        in_specs=[blk, blk],
        out_specs=blk,
        out_shape=jax.ShapeDtypeStruct((NPAD, D), jnp.float32),
    )(a, b)


def _tc_loss(rows, uid_c, uid_r, pid_c, pid_r):
    RB = 512

    def body(rows_ref, uidc_ref, uidr_ref, pidc_ref, pidr_ref, o_ref,
             un_ref, pn_ref, wu_ref, wp_ref):
        def norm(x):
            return x / (jnp.sqrt(jnp.sum(x * x, axis=1, keepdims=True)) + 1e-12)

        ue = norm(rows_ref[pl.ds(0, B), :])
        ie = norm(rows_ref[pl.ds(B, B), :])
        un_ref[...] = ue.astype(jnp.bfloat16)
        pn_ref[...] = ie.astype(jnp.bfloat16)

        diff = ue - ie
        d = jnp.sqrt(jnp.sum(diff * diff, axis=1))
        t = d + 1e-12
        align = jnp.sum(t * t) / B

        def first_occurrence(idc_ref, idr_ref, w_ref):
            ids_row = idr_ref[...]

            def blkstep(k, _):
                idb = idc_ref[pl.ds(k * RB, RB), :]
                eq = idb == ids_row
                col = lax.broadcasted_iota(jnp.int32, (RB, B), 1)
                row = lax.broadcasted_iota(jnp.int32, (RB, B), 0) + k * RB
                dup = jnp.max(jnp.where(eq & (col < row), 1.0, 0.0),
                              axis=1, keepdims=True)
                w_ref[pl.ds(k * RB, RB), :] = 1.0 - dup
                return 0

            lax.fori_loop(0, B // RB, blkstep, 0)

        first_occurrence(uidc_ref, uidr_ref, wu_ref)
        first_occurrence(pidc_ref, pidr_ref, wp_ref)

        def uniform(xn_ref, wc_ref):
            w_full = wc_ref[...]

            def blkstep(k, s):
                xb = xn_ref[pl.ds(k * RB, RB), :]
                g = lax.dot_general(xb, xn_ref[...],
                                    (((1,), (1,)), ((), ())),
                                    preferred_element_type=jnp.float32)
                sq = jnp.maximum(2.0 - 2.0 * g, 0.0)
                e = jnp.exp(-T_CONST * sq)
                ew = lax.dot_general(e, w_full, (((1,), (0,)), ((), ())),
                                     preferred_element_type=jnp.float32)
                wc = wc_ref[pl.ds(k * RB, RB), :]
                return s + jnp.sum(ew * wc)

            s = lax.fori_loop(0, B // RB, blkstep, 0.0)
            n = jnp.sum(w_full)
            return jnp.log((s - n) / (n * (n - 1.0)) + 1e-12)

        lu = uniform(un_ref, wu_ref)
        lp = uniform(pn_ref, wp_ref)
        uni = GAMMA * (lu + lp) / 2.0

        li = lax.broadcasted_iota(jnp.int32, (1, 128), 1)
        o_ref[...] = jnp.where(li == 0, align,
                               jnp.where(li == 1, uni, 0.0))

    return pl.pallas_call(
        body,
        out_shape=jax.ShapeDtypeStruct((1, 128), jnp.float32),
        scratch_shapes=[
            pltpu.VMEM((B, D), jnp.bfloat16),
            pltpu.VMEM((B, D), jnp.bfloat16),
            pltpu.VMEM((B, 1), jnp.float32),
            pltpu.VMEM((B, 1), jnp.float32),
        ],
    )(rows, uid_c, uid_r, pid_c, pid_r)


def kernel(user, positive, adj_row, adj_col, adj_val, user_table, item_table):
    user = user.astype(jnp.int32)
    positive = positive.astype(jnp.int32)
    adj_row = adj_row.astype(jnp.int32)
    adj_col = adj_col.astype(jnp.int32)

    emb0 = jnp.zeros((NPAD, D), jnp.float32)
    emb0 = emb0.at[:N_USERS].set(user_table).at[N_USERS:N].set(item_table)

    p1a, p1b = _spmm_kernel(adj_row, adj_col, adj_val, emb0)
    emb1 = _tc_add2(p1a, p1b)
    p2a, p2b = _spmm_kernel(adj_row, adj_col, adj_val, emb1)

    cat_idx = jnp.concatenate([user, N_USERS + positive])
    rows = _sc_gather3(emb1, p2a, p2b, cat_idx, 2 * B)

    o = _tc_loss(rows, user.reshape(B, 1), user.reshape(1, B),
                 positive.reshape(B, 1), positive.reshape(1, B))
    return jnp.stack([o[0, 0], o[0, 1]])


# speedup vs baseline: 1.0042x; 1.0042x over previous
import functools

import jax
import jax.numpy as jnp
from jax import lax
from jax.experimental import pallas as pl
from jax.experimental.pallas import tpu as pltpu
from jax.experimental.pallas import tpu_sc as plsc

N_USERS = 30000
N_ITEMS = 20000
N = N_USERS + N_ITEMS
D = 32
NNZ = 1600000
B = 4096
T_CONST = 2.0
GAMMA = 1.0

NPAD = 50048
NC, NS, L = 2, 16, 16
NW = NC * NS
EPW = NNZ // NW
SUB = 80
KSUB = 5
CHUNK = SUB * KSUB
NCH = EPW // CHUNK
ZROWS = 391


def _spmm_kernel(adj_row, adj_col, adj_val, emb):
    mesh = plsc.VectorSubcoreMesh(core_axis_name="c", subcore_axis_name="s")

    @functools.partial(
        pl.kernel,
        mesh=mesh,
        out_type=(
            jax.ShapeDtypeStruct((NPAD, D), jnp.float32),
            jax.ShapeDtypeStruct((NPAD, D), jnp.float32),
        ),
        scratch_types=[
            pltpu.VMEM((2, CHUNK), jnp.int32),
            pltpu.VMEM((2, CHUNK), jnp.int32),
            pltpu.VMEM((2, CHUNK), jnp.float32),
            pltpu.VMEM((2, KSUB, SUB), jnp.int32),
            pltpu.VMEM((CHUNK, D), jnp.float32),
            pltpu.VMEM((CHUNK, D), jnp.float32),
            pltpu.VMEM_SHARED((NPAD, D), jnp.float32),
            pltpu.SemaphoreType.DMA,
            pltpu.SemaphoreType.DMA,
            pltpu.SemaphoreType.DMA,
            pltpu.SemaphoreType.DMA,
            pltpu.SemaphoreType.DMA,
            pltpu.SemaphoreType.DMA,
        ],
        compiler_params=pltpu.CompilerParams(use_tc_tiling_on_sc=False),
    )
    def k(rows_h, cols_h, vals_h, emb_h, out0, out1, colv, rowv, valv, rsc,
          gbuf0, gbuf1, accum, sl0, sl1, sg0, sg1, ss0, ss1):
        cid = lax.axis_index("c")
        sid = lax.axis_index("s")
        wid = cid * NS + sid
        gbufs = (gbuf0, gbuf1)
        sls = (sl0, sl1)
        sgs = (sg0, sg1)
        sss = (ss0, ss1)

        zeros16 = jnp.zeros((L,), jnp.float32)

        def zrow(i, _):
            gbuf0[i, pl.ds(0, L)] = zeros16
            gbuf0[i, pl.ds(L, L)] = zeros16
            return 0

        lax.fori_loop(0, ZROWS, zrow, 0)

        def zcopy(q, _):
            pltpu.sync_copy(gbuf0.at[pl.ds(0, ZROWS)],
                            accum.at[pl.ds(sid * (NPAD // NS) + q * ZROWS, ZROWS)])
            return 0

        lax.fori_loop(0, (NPAD // NS) // ZROWS, zcopy, 0)
        plsc.subcore_barrier()

        ebase = wid * EPW

        def fire_loads(c, b):
            e0 = ebase + c * CHUNK
            pltpu.async_copy(cols_h.at[pl.ds(e0, CHUNK)], colv.at[b], sls[b])
            pltpu.async_copy(rows_h.at[pl.ds(e0, CHUNK)], rowv.at[b], sls[b])
            pltpu.async_copy(vals_h.at[pl.ds(e0, CHUNK)], valv.at[b], sls[b])

        def drain_loads(b):
            pltpu.make_async_copy(cols_h.at[pl.ds(0, CHUNK)], colv.at[b],
                                  sls[b]).wait()
            pltpu.make_async_copy(rows_h.at[pl.ds(0, CHUNK)], rowv.at[b],
                                  sls[b]).wait()
            pltpu.make_async_copy(vals_h.at[pl.ds(0, CHUNK)], valv.at[b],
                                  sls[b]).wait()

        def fire_gathers(b):
            for j in range(KSUB):
                pltpu.async_copy(emb_h.at[colv.at[b, pl.ds(j * SUB, SUB)]],
                                 gbufs[b].at[pl.ds(j * SUB, SUB)], sgs[b])

        def drain_gathers(b):
            for j in range(KSUB):
                pltpu.make_async_copy(emb_h.at[pl.ds(0, SUB)],
                                      gbufs[b].at[pl.ds(j * SUB, SUB)],
                                      sgs[b]).wait()

        def scale(b):
            g_ref = gbufs[b]
            for j in range(KSUB):
                def grp(g, _, j=j):
                    v16 = valv[b, pl.ds(j * SUB + g * L, L)]
                    for u in range(L):
                        e = j * SUB + g * L + u
                        vb = v16.at[jnp.full((L,), u, jnp.int32)].get(
                            mode="promise_in_bounds")
                        a = g_ref[e, pl.ds(0, L)] * vb
                        bb = g_ref[e, pl.ds(L, L)] * vb
                        g_ref[e, pl.ds(0, L)] = a
                        g_ref[e, pl.ds(L, L)] = bb
                    return 0

                lax.fori_loop(0, SUB // L, grp, 0)

        def copy_scatter_idx(b):
            for j in range(KSUB):
                def cp(g, _, j=j):
                    rsc[b, j, pl.ds(g * L, L)] = rowv[b, pl.ds(j * SUB + g * L, L)]
                    return 0

                lax.fori_loop(0, SUB // L, cp, 0)

        def fire_scatter(b):
            for j in range(KSUB):
                pltpu.async_copy(gbufs[b].at[pl.ds(j * SUB, SUB)],
                                 accum.at[rsc.at[b, j]], sss[b], add=True)

        def drain_scatter(b):
            for j in range(KSUB):
                pltpu.make_async_copy(gbufs[b].at[pl.ds(j * SUB, SUB)],
                                      accum.at[pl.ds(0, SUB)], sss[b]).wait()

        def step(c, b, first, last):
            drain_gathers(b)
            drain_loads(1 - b)
            if not first:
                drain_scatter(1 - b)
            fire_gathers(1 - b)
            copy_scatter_idx(b)
            scale(b)
            fire_scatter(b)
            r_next = jnp.minimum(c + 2, NCH - 1)
            fire_loads(r_next, b)

        fire_loads(0, 0)
        drain_loads(0)
        fire_gathers(0)
        fire_loads(1, 1)
        step(0, 0, first=True, last=False)
        step(1, 1, first=False, last=False)

        def pair(p, _):
            for b in (0, 1):
                step(2 * p + 2 + b, b, first=False, last=False)
            return 0

        lax.fori_loop(0, (NCH - 3) // 2, pair, 0)

        drain_gathers(0)
        drain_scatter(1)
        copy_scatter_idx(0)
        scale(0)
        fire_scatter(0)
        drain_scatter(0)
        drain_loads(1)
        plsc.subcore_barrier()

        def rd(q, _):
            r0 = sid * (NPAD // NS) + q * ZROWS
            pltpu.sync_copy(accum.at[pl.ds(r0, ZROWS)], gbuf0.at[pl.ds(0, ZROWS)])

            @pl.when(cid == 0)
            def _():
                pltpu.sync_copy(gbuf0.at[pl.ds(0, ZROWS)], out0.at[pl.ds(r0, ZROWS)])

            @pl.when(cid == 1)
            def _():
                pltpu.sync_copy(gbuf0.at[pl.ds(0, ZROWS)], out1.at[pl.ds(r0, ZROWS)])
            return 0

        lax.fori_loop(0, (NPAD // NS) // ZROWS, rd, 0)

    return k(adj_row, adj_col, adj_val, emb)


def _sc_gather3(t0, t1, t2, idx, nrows):
    mesh = plsc.VectorSubcoreMesh(core_axis_name="c", subcore_axis_name="s")
    per_w = nrows // NW
    kq = per_w // 128

    @functools.partial(
        pl.kernel,
        mesh=mesh,
        out_type=jax.ShapeDtypeStruct((nrows, D), jnp.float32),
        scratch_types=[
            pltpu.VMEM((per_w,), jnp.int32),
            pltpu.VMEM((per_w, D), jnp.float32),
            pltpu.VMEM((per_w, D), jnp.float32),
            pltpu.VMEM((per_w, D), jnp.float32),
            pltpu.SemaphoreType.DMA,
        ],
        compiler_params=pltpu.CompilerParams(use_tc_tiling_on_sc=False),
    )
    def k(t0_h, t1_h, t2_h, idx_h, out_h, idxv, b0, b1, b2, sem):
        wid = lax.axis_index("c") * NS + lax.axis_index("s")
        pltpu.sync_copy(idx_h.at[pl.ds(wid * per_w, per_w)], idxv)
        copies = []
        for tab, buf in ((t0_h, b0), (t1_h, b1), (t2_h, b2)):
            for q in range(kq):
                copies.append(
                    pltpu.async_copy(tab.at[idxv.at[pl.ds(q * 128, 128)]],
                                     buf.at[pl.ds(q * 128, 128)], sem))
        for cpy in copies:
            cpy.wait()

        def addrow(r, _):
            a0 = b0[r, pl.ds(0, L)] + b1[r, pl.ds(0, L)] + b2[r, pl.ds(0, L)]
            a1 = b0[r, pl.ds(L, L)] + b1[r, pl.ds(L, L)] + b2[r, pl.ds(L, L)]
            b0[r, pl.ds(0, L)] = a0
            b0[r, pl.ds(L, L)] = a1
            return 0

        lax.fori_loop(0, per_w, addrow, 0)
        pltpu.sync_copy(b0, out_h.at[pl.ds(wid * per_w, per_w)])

    return k(t0, t1, t2, idx)


def _tc_add2(a, b):
    def body(a_ref, b_ref, o_ref):
        o_ref[...] = a_ref[...] + b_ref[...]

    blk = pl.BlockSpec((NPAD // 16, D), lambda i: (i, 0))
    return pl.pallas_call(
        body,
        grid=(16,),
        in_specs=[blk, blk],
        out_specs=blk,
        out_shape=jax.ShapeDtypeStruct((NPAD, D), jnp.float32),
    )(a, b)


def _tc_loss(rows, wu_c, wp_c):
    RB = 512

    def body(rows_ref, wuc_ref, wpc_ref, o_ref, un_ref, pn_ref):
        def norm(x):
            return x / (jnp.sqrt(jnp.sum(x * x, axis=1, keepdims=True)) + 1e-12)

        ue = norm(rows_ref[pl.ds(0, B), :])
        ie = norm(rows_ref[pl.ds(B, B), :])
        un_ref[...] = ue.astype(jnp.bfloat16)
        pn_ref[...] = ie.astype(jnp.bfloat16)

        diff = ue - ie
        d = jnp.sqrt(jnp.sum(diff * diff, axis=1))
        t = d + 1e-12
        align = jnp.sum(t * t) / B

        def uniform(xn_ref, wc_ref):
            w_full = wc_ref[...]

            def blkstep(k, s):
                xb = xn_ref[pl.ds(k * RB, RB), :]
                g = lax.dot_general(xb, xn_ref[...],
                                    (((1,), (1,)), ((), ())),
                                    preferred_element_type=jnp.float32)
                sq = jnp.maximum(2.0 - 2.0 * g, 0.0)
                e = jnp.exp(-T_CONST * sq)
                ew = lax.dot_general(e, w_full, (((1,), (0,)), ((), ())),
                                     preferred_element_type=jnp.float32)
                wc = wc_ref[pl.ds(k * RB, RB), :]
                return s + jnp.sum(ew * wc)

            s = lax.fori_loop(0, B // RB, blkstep, 0.0)
            n = jnp.sum(w_full)
            return jnp.log((s - n) / (n * (n - 1.0)) + 1e-12)

        lu = uniform(un_ref, wuc_ref)
        lp = uniform(pn_ref, wpc_ref)
        uni = GAMMA * (lu + lp) / 2.0

        li = lax.broadcasted_iota(jnp.int32, (1, 128), 1)
        o_ref[...] = jnp.where(li == 0, align,
                               jnp.where(li == 1, uni, 0.0))

    return pl.pallas_call(
        body,
        out_shape=jax.ShapeDtypeStruct((1, 128), jnp.float32),
        scratch_shapes=[
            pltpu.VMEM((B, D), jnp.bfloat16),
            pltpu.VMEM((B, D), jnp.bfloat16),
        ],
    )(rows, wu_c, wp_c)


def kernel(user, positive, adj_row, adj_col, adj_val, user_table, item_table):
    user = user.astype(jnp.int32)
    positive = positive.astype(jnp.int32)
    adj_row = adj_row.astype(jnp.int32)
    adj_col = adj_col.astype(jnp.int32)

    emb0 = jnp.concatenate([user_table, item_table], axis=0)

    p1a, p1b = _spmm_kernel(adj_row, adj_col, adj_val, emb0)
    emb1 = _tc_add2(p1a, p1b)
    p2a, p2b = _spmm_kernel(adj_row, adj_col, adj_val, emb1)

    cat_idx = jnp.concatenate([user, N_USERS + positive])
    rows = _sc_gather3(emb1, p2a, p2b, cat_idx, 2 * B)

    def first_occ_weights(ids):
        perm = jnp.argsort(ids)
        s = ids[perm]
        first = jnp.concatenate(
            [jnp.ones((1,), jnp.float32), (s[1:] != s[:-1]).astype(jnp.float32)])
        return jnp.zeros((B,), jnp.float32).at[perm].set(first)

    wu = first_occ_weights(user)
    wp = first_occ_weights(positive)
    o = _tc_loss(rows, wu.reshape(B, 1), wp.reshape(B, 1))
    return jnp.stack([o[0, 0], o[0, 1]])


# speedup vs baseline: 1.1316x; 1.1269x over previous
import functools

import jax
import jax.numpy as jnp
from jax import lax
from jax.experimental import pallas as pl
from jax.experimental.pallas import tpu as pltpu
from jax.experimental.pallas import tpu_sc as plsc

N_USERS = 30000
N_ITEMS = 20000
N = N_USERS + N_ITEMS
D = 32
NNZ = 1600000
B = 4096
T_CONST = 2.0
GAMMA = 1.0

NPAD = 50048
NC, NS, L = 2, 16, 16
NW = NC * NS
EPW = NNZ // NW
SUB = 80
KSUB = 5
CHUNK = SUB * KSUB
NCH = EPW // CHUNK
ZROWS = 391


def _spmm_kernel(adj_row, adj_col, adj_val, emb):
    mesh = plsc.VectorSubcoreMesh(core_axis_name="c", subcore_axis_name="s")

    @functools.partial(
        pl.kernel,
        mesh=mesh,
        out_type=(
            jax.ShapeDtypeStruct((NPAD, D), jnp.float32),
            jax.ShapeDtypeStruct((NPAD, D), jnp.float32),
        ),
        scratch_types=[
            pltpu.VMEM((2, CHUNK), jnp.int32),
            pltpu.VMEM((2, CHUNK), jnp.int32),
            pltpu.VMEM((2, CHUNK), jnp.float32),
            pltpu.VMEM((2, KSUB, SUB), jnp.int32),
            pltpu.VMEM((CHUNK, D), jnp.float32),
            pltpu.VMEM((CHUNK, D), jnp.float32),
            pltpu.VMEM_SHARED((NPAD, D), jnp.float32),
            pltpu.SemaphoreType.DMA,
            pltpu.SemaphoreType.DMA,
            pltpu.SemaphoreType.DMA,
            pltpu.SemaphoreType.DMA,
            pltpu.SemaphoreType.DMA,
            pltpu.SemaphoreType.DMA,
        ],
        compiler_params=pltpu.CompilerParams(use_tc_tiling_on_sc=False),
    )
    def k(rows_h, cols_h, vals_h, emb_h, out0, out1, colv, rowv, valv, rsc,
          gbuf0, gbuf1, accum, sl0, sl1, sg0, sg1, ss0, ss1):
        cid = lax.axis_index("c")
        sid = lax.axis_index("s")
        wid = cid * NS + sid
        gbufs = (gbuf0, gbuf1)
        sls = (sl0, sl1)
        sgs = (sg0, sg1)
        sss = (ss0, ss1)

        zeros16 = jnp.zeros((L,), jnp.float32)

        def zrow(i, _):
            gbuf0[i, pl.ds(0, L)] = zeros16
            gbuf0[i, pl.ds(L, L)] = zeros16
            return 0

        lax.fori_loop(0, ZROWS, zrow, 0)

        def zcopy(q, _):
            pltpu.sync_copy(gbuf0.at[pl.ds(0, ZROWS)],
                            accum.at[pl.ds(sid * (NPAD // NS) + q * ZROWS, ZROWS)])
            return 0

        lax.fori_loop(0, (NPAD // NS) // ZROWS, zcopy, 0)
        plsc.subcore_barrier()

        ebase = wid * EPW

        def fire_loads(c, b):
            e0 = ebase + c * CHUNK
            pltpu.async_copy(cols_h.at[pl.ds(e0, CHUNK)], colv.at[b], sls[b])
            pltpu.async_copy(rows_h.at[pl.ds(e0, CHUNK)], rowv.at[b], sls[b])
            pltpu.async_copy(vals_h.at[pl.ds(e0, CHUNK)], valv.at[b], sls[b])

        def drain_loads(b):
            pltpu.make_async_copy(cols_h.at[pl.ds(0, CHUNK)], colv.at[b],
                                  sls[b]).wait()
            pltpu.make_async_copy(rows_h.at[pl.ds(0, CHUNK)], rowv.at[b],
                                  sls[b]).wait()
            pltpu.make_async_copy(vals_h.at[pl.ds(0, CHUNK)], valv.at[b],
                                  sls[b]).wait()

        def fire_gathers(b):
            for j in range(KSUB):
                pltpu.async_copy(emb_h.at[colv.at[b, pl.ds(j * SUB, SUB)]],
                                 gbufs[b].at[pl.ds(j * SUB, SUB)], sgs[b])

        def drain_gathers(b):
            for j in range(KSUB):
                pltpu.make_async_copy(emb_h.at[pl.ds(0, SUB)],
                                      gbufs[b].at[pl.ds(j * SUB, SUB)],
                                      sgs[b]).wait()

        def scale(b):
            g_ref = gbufs[b]
            for j in range(KSUB):
                def grp(g, _, j=j):
                    v16 = valv[b, pl.ds(j * SUB + g * L, L)]
                    for u in range(L):
                        e = j * SUB + g * L + u
                        vb = v16.at[jnp.full((L,), u, jnp.int32)].get(
                            mode="promise_in_bounds")
                        a = g_ref[e, pl.ds(0, L)] * vb
                        bb = g_ref[e, pl.ds(L, L)] * vb
                        g_ref[e, pl.ds(0, L)] = a
                        g_ref[e, pl.ds(L, L)] = bb
                    return 0

                lax.fori_loop(0, SUB // L, grp, 0)

        def copy_scatter_idx(b):
            for j in range(KSUB):
                def cp(g, _, j=j):
                    rsc[b, j, pl.ds(g * L, L)] = rowv[b, pl.ds(j * SUB + g * L, L)]
                    return 0

                lax.fori_loop(0, SUB // L, cp, 0)

        def fire_scatter(b):
            for j in range(KSUB):
                pltpu.async_copy(gbufs[b].at[pl.ds(j * SUB, SUB)],
                                 accum.at[rsc.at[b, j]], sss[b], add=True)

        def drain_scatter(b):
            for j in range(KSUB):
                pltpu.make_async_copy(gbufs[b].at[pl.ds(j * SUB, SUB)],
                                      accum.at[pl.ds(0, SUB)], sss[b]).wait()

        def step(c, b, first, last):
            drain_gathers(b)
            drain_loads(1 - b)
            if not first:
                drain_scatter(1 - b)
            fire_gathers(1 - b)
            copy_scatter_idx(b)
            scale(b)
            fire_scatter(b)
            r_next = jnp.minimum(c + 2, NCH - 1)
            fire_loads(r_next, b)

        fire_loads(0, 0)
        drain_loads(0)
        fire_gathers(0)
        fire_loads(1, 1)
        step(0, 0, first=True, last=False)
        step(1, 1, first=False, last=False)

        def pair(p, _):
            for b in (0, 1):
                step(2 * p + 2 + b, b, first=False, last=False)
            return 0

        lax.fori_loop(0, (NCH - 3) // 2, pair, 0)

        drain_gathers(0)
        drain_scatter(1)
        copy_scatter_idx(0)
        scale(0)
        fire_scatter(0)
        drain_scatter(0)
        drain_loads(1)
        plsc.subcore_barrier()

        def rd(q, _):
            r0 = sid * (NPAD // NS) + q * ZROWS
            pltpu.sync_copy(accum.at[pl.ds(r0, ZROWS)], gbuf0.at[pl.ds(0, ZROWS)])

            @pl.when(cid == 0)
            def _():
                pltpu.sync_copy(gbuf0.at[pl.ds(0, ZROWS)], out0.at[pl.ds(r0, ZROWS)])

            @pl.when(cid == 1)
            def _():
                pltpu.sync_copy(gbuf0.at[pl.ds(0, ZROWS)], out1.at[pl.ds(r0, ZROWS)])
            return 0

        lax.fori_loop(0, (NPAD // NS) // ZROWS, rd, 0)

    return k(adj_row, adj_col, adj_val, emb)


def _sc_gather3(t0, t1, t2, idx, nrows):
    mesh = plsc.VectorSubcoreMesh(core_axis_name="c", subcore_axis_name="s")
    per_w = nrows // NW
    kq = per_w // 128

    @functools.partial(
        pl.kernel,
        mesh=mesh,
        out_type=jax.ShapeDtypeStruct((nrows, D), jnp.float32),
        scratch_types=[
            pltpu.VMEM((per_w,), jnp.int32),
            pltpu.VMEM((per_w, D), jnp.float32),
            pltpu.VMEM((per_w, D), jnp.float32),
            pltpu.VMEM((per_w, D), jnp.float32),
            pltpu.SemaphoreType.DMA,
        ],
        compiler_params=pltpu.CompilerParams(use_tc_tiling_on_sc=False),
    )
    def k(t0_h, t1_h, t2_h, idx_h, out_h, idxv, b0, b1, b2, sem):
        wid = lax.axis_index("c") * NS + lax.axis_index("s")
        pltpu.sync_copy(idx_h.at[pl.ds(wid * per_w, per_w)], idxv)
        copies = []
        for tab, buf in ((t0_h, b0), (t1_h, b1), (t2_h, b2)):
            for q in range(kq):
                copies.append(
                    pltpu.async_copy(tab.at[idxv.at[pl.ds(q * 128, 128)]],
                                     buf.at[pl.ds(q * 128, 128)], sem))
        for cpy in copies:
            cpy.wait()

        def addrow(r, _):
            a0 = b0[r, pl.ds(0, L)] + b1[r, pl.ds(0, L)] + b2[r, pl.ds(0, L)]
            a1 = b0[r, pl.ds(L, L)] + b1[r, pl.ds(L, L)] + b2[r, pl.ds(L, L)]
            b0[r, pl.ds(0, L)] = a0
            b0[r, pl.ds(L, L)] = a1
            return 0

        lax.fori_loop(0, per_w, addrow, 0)
        pltpu.sync_copy(b0, out_h.at[pl.ds(wid * per_w, per_w)])

    return k(t0, t1, t2, idx)


def _sc_add2(a, b):
    mesh = plsc.VectorSubcoreMesh(core_axis_name="c", subcore_axis_name="s")
    RPW = NPAD // NW
    RC = 391

    @functools.partial(
        pl.kernel,
        mesh=mesh,
        out_type=jax.ShapeDtypeStruct((NPAD, D), jnp.float32),
        scratch_types=[
            pltpu.VMEM((RC, D), jnp.float32),
            pltpu.VMEM((RC, D), jnp.float32),
            pltpu.SemaphoreType.DMA,
        ],
        compiler_params=pltpu.CompilerParams(use_tc_tiling_on_sc=False),
    )
    def k(a_h, b_h, o_h, abuf, bbuf, sem):
        wid = lax.axis_index("c") * NS + lax.axis_index("s")

        def chunk(q, _):
            r0 = wid * RPW + q * RC
            ca = pltpu.async_copy(a_h.at[pl.ds(r0, RC)], abuf, sem)
            cb = pltpu.async_copy(b_h.at[pl.ds(r0, RC)], bbuf, sem)
            ca.wait()
            cb.wait()

            def addrow(r, _):
                abuf[r, pl.ds(0, L)] = abuf[r, pl.ds(0, L)] + bbuf[r, pl.ds(0, L)]
                abuf[r, pl.ds(L, L)] = abuf[r, pl.ds(L, L)] + bbuf[r, pl.ds(L, L)]
                return 0

            lax.fori_loop(0, RC, addrow, 0)
            pltpu.sync_copy(abuf, o_h.at[pl.ds(r0, RC)])
            return 0

        lax.fori_loop(0, RPW // RC, chunk, 0)

    return k(a, b)


def _sc_concat_tables(user_table, item_table):
    mesh = plsc.VectorSubcoreMesh(core_axis_name="c", subcore_axis_name="s")
    UPW = N_USERS // NS
    IPW = N_ITEMS // NS
    RC = 625

    @functools.partial(
        pl.kernel,
        mesh=mesh,
        out_type=jax.ShapeDtypeStruct((N, D), jnp.float32),
        scratch_types=[
            pltpu.VMEM((RC, D), jnp.float32),
            pltpu.SemaphoreType.DMA,
        ],
        compiler_params=pltpu.CompilerParams(use_tc_tiling_on_sc=False),
    )
    def k(u_h, i_h, o_h, buf, sem):
        cid = lax.axis_index("c")
        sid = lax.axis_index("s")

        @pl.when(cid == 0)
        def _():
            def chunk(q, _):
                r0 = sid * UPW + q * RC
                pltpu.async_copy(u_h.at[pl.ds(r0, RC)], buf, sem).wait()
                pltpu.sync_copy(buf, o_h.at[pl.ds(r0, RC)])
                return 0

            lax.fori_loop(0, UPW // RC, chunk, 0)

        @pl.when(cid == 1)
        def _():
            def chunk(q, _):
                r0 = sid * IPW + q * RC
                pltpu.async_copy(i_h.at[pl.ds(r0, RC)], buf, sem).wait()
                pltpu.sync_copy(buf, o_h.at[pl.ds(N_USERS + r0, RC)])
                return 0

            lax.fori_loop(0, IPW // RC, chunk, 0)

    return k(user_table, item_table)


def _tc_loss(rows, wu_c, wp_c):
    RB = 512

    def body(rows_ref, wuc_ref, wpc_ref, o_ref, un_ref, pn_ref):
        def norm(x):
            return x / (jnp.sqrt(jnp.sum(x * x, axis=1, keepdims=True)) + 1e-12)

        ue = norm(rows_ref[pl.ds(0, B), :])
        ie = norm(rows_ref[pl.ds(B, B), :])
        un_ref[...] = norm(rows_ref[pl.ds(2 * B, B), :]).astype(jnp.bfloat16)
        pn_ref[...] = norm(rows_ref[pl.ds(3 * B, B), :]).astype(jnp.bfloat16)

        diff = ue - ie
        d = jnp.sqrt(jnp.sum(diff * diff, axis=1))
        t = d + 1e-12
        align = jnp.sum(t * t) / B

        def uniform(xn_ref, wc_ref):
            w_full = wc_ref[...]

            def blkstep(k, s):
                xb = xn_ref[pl.ds(k * RB, RB), :]
                g = lax.dot_general(xb, xn_ref[...],
                                    (((1,), (1,)), ((), ())),
                                    preferred_element_type=jnp.float32)
                sq = jnp.maximum(2.0 - 2.0 * g, 0.0)
                e = jnp.exp(-T_CONST * sq)
                ew = lax.dot_general(e, w_full, (((1,), (0,)), ((), ())),
                                     preferred_element_type=jnp.float32)
                wc = wc_ref[pl.ds(k * RB, RB), :]
                return s + jnp.sum(ew * wc)

            s = lax.fori_loop(0, B // RB, blkstep, 0.0)
            n = jnp.sum(w_full)
            return jnp.log((s - n) / (n * (n - 1.0)) + 1e-12)

        lu = uniform(un_ref, wuc_ref)
        lp = uniform(pn_ref, wpc_ref)
        uni = GAMMA * (lu + lp) / 2.0

        li = lax.broadcasted_iota(jnp.int32, (1, 128), 1)
        o_ref[...] = jnp.where(li == 0, align,
                               jnp.where(li == 1, uni, 0.0))

    return pl.pallas_call(
        body,
        out_shape=jax.ShapeDtypeStruct((1, 128), jnp.float32),
        scratch_shapes=[
            pltpu.VMEM((B, D), jnp.bfloat16),
            pltpu.VMEM((B, D), jnp.bfloat16),
        ],
    )(rows, wu_c, wp_c)


def kernel(user, positive, adj_row, adj_col, adj_val, user_table, item_table):
    user = user.astype(jnp.int32)
    positive = positive.astype(jnp.int32)
    adj_row = adj_row.astype(jnp.int32)
    adj_col = adj_col.astype(jnp.int32)

    emb0 = _sc_concat_tables(user_table, item_table)

    p1a, p1b = _spmm_kernel(adj_row, adj_col, adj_val, emb0)
    emb1 = _sc_add2(p1a, p1b)
    p2a, p2b = _spmm_kernel(adj_row, adj_col, adj_val, emb1)

    su = jnp.sort(user)
    sp = jnp.sort(positive)
    cat_idx = jnp.concatenate([user, N_USERS + positive, su, N_USERS + sp])
    rows = _sc_gather3(emb1, p2a, p2b, cat_idx, 4 * B)

    wu = jnp.concatenate(
        [jnp.ones((1,), jnp.float32), (su[1:] != su[:-1]).astype(jnp.float32)])
    wp = jnp.concatenate(
        [jnp.ones((1,), jnp.float32), (sp[1:] != sp[:-1]).astype(jnp.float32)])
    o = _tc_loss(rows, wu.reshape(B, 1), wp.reshape(B, 1))
    return jnp.stack([o[0, 0], o[0, 1]])


# speedup vs baseline: 1.1507x; 1.0169x over previous
import functools

import jax
import jax.numpy as jnp
from jax import lax
from jax.experimental import pallas as pl
from jax.experimental.pallas import tpu as pltpu
from jax.experimental.pallas import tpu_sc as plsc

N_USERS = 30000
N_ITEMS = 20000
N = N_USERS + N_ITEMS
D = 32
NNZ = 1600000
B = 4096
T_CONST = 2.0
GAMMA = 1.0

NPAD = 50048
NC, NS, L = 2, 16, 16
NW = NC * NS
EPW = NNZ // NW
SUB = 80
KSUB = 5
CHUNK = SUB * KSUB
NCH = EPW // CHUNK
ZROWS = 391


def _spmm_kernel(adj_row, adj_col, adj_val, emb):
    mesh = plsc.VectorSubcoreMesh(core_axis_name="c", subcore_axis_name="s")

    @functools.partial(
        pl.kernel,
        mesh=mesh,
        out_type=(
            jax.ShapeDtypeStruct((NPAD, D), jnp.float32),
            jax.ShapeDtypeStruct((NPAD, D), jnp.float32),
        ),
        scratch_types=[
            pltpu.VMEM((2, CHUNK), jnp.int32),
            pltpu.VMEM((2, CHUNK), jnp.int32),
            pltpu.VMEM((2, CHUNK), jnp.float32),
            pltpu.VMEM((2, KSUB, SUB), jnp.int32),
            pltpu.VMEM((CHUNK, D), jnp.float32),
            pltpu.VMEM((CHUNK, D), jnp.float32),
            pltpu.VMEM_SHARED((NPAD, D), jnp.float32),
            pltpu.SemaphoreType.DMA,
            pltpu.SemaphoreType.DMA,
            pltpu.SemaphoreType.DMA,
            pltpu.SemaphoreType.DMA,
            pltpu.SemaphoreType.DMA,
            pltpu.SemaphoreType.DMA,
        ],
        compiler_params=pltpu.CompilerParams(use_tc_tiling_on_sc=False),
    )
    def k(rows_h, cols_h, vals_h, emb_h, out0, out1, colv, rowv, valv, rsc,
          gbuf0, gbuf1, accum, sl0, sl1, sg0, sg1, ss0, ss1):
        cid = lax.axis_index("c")
        sid = lax.axis_index("s")
        wid = cid * NS + sid
        gbufs = (gbuf0, gbuf1)
        sls = (sl0, sl1)
        sgs = (sg0, sg1)
        sss = (ss0, ss1)

        zeros16 = jnp.zeros((L,), jnp.float32)

        def zrow(i, _):
            gbuf0[i, pl.ds(0, L)] = zeros16
            gbuf0[i, pl.ds(L, L)] = zeros16
            return 0

        lax.fori_loop(0, ZROWS, zrow, 0)

        def zcopy(q, _):
            pltpu.sync_copy(gbuf0.at[pl.ds(0, ZROWS)],
                            accum.at[pl.ds(sid * (NPAD // NS) + q * ZROWS, ZROWS)])
            return 0

        lax.fori_loop(0, (NPAD // NS) // ZROWS, zcopy, 0)
        plsc.subcore_barrier()

        ebase = wid * EPW

        def fire_loads(c, b):
            e0 = ebase + c * CHUNK
            pltpu.async_copy(cols_h.at[pl.ds(e0, CHUNK)], colv.at[b], sls[b])
            pltpu.async_copy(rows_h.at[pl.ds(e0, CHUNK)], rowv.at[b], sls[b])
            pltpu.async_copy(vals_h.at[pl.ds(e0, CHUNK)], valv.at[b], sls[b])

        def drain_loads(b):
            pltpu.make_async_copy(cols_h.at[pl.ds(0, CHUNK)], colv.at[b],
                                  sls[b]).wait()
            pltpu.make_async_copy(rows_h.at[pl.ds(0, CHUNK)], rowv.at[b],
                                  sls[b]).wait()
            pltpu.make_async_copy(vals_h.at[pl.ds(0, CHUNK)], valv.at[b],
                                  sls[b]).wait()

        def fire_gathers(b):
            for j in range(KSUB):
                pltpu.async_copy(emb_h.at[colv.at[b, pl.ds(j * SUB, SUB)]],
                                 gbufs[b].at[pl.ds(j * SUB, SUB)], sgs[b])

        def drain_gathers(b):
            for j in range(KSUB):
                pltpu.make_async_copy(emb_h.at[pl.ds(0, SUB)],
                                      gbufs[b].at[pl.ds(j * SUB, SUB)],
                                      sgs[b]).wait()

        def scale(b):
            g_ref = gbufs[b]
            for j in range(KSUB):
                def grp(g, _, j=j):
                    v16 = valv[b, pl.ds(j * SUB + g * L, L)]
                    for u in range(L):
                        e = j * SUB + g * L + u
                        vb = v16.at[jnp.full((L,), u, jnp.int32)].get(
                            mode="promise_in_bounds")
                        a = g_ref[e, pl.ds(0, L)] * vb
                        bb = g_ref[e, pl.ds(L, L)] * vb
                        g_ref[e, pl.ds(0, L)] = a
                        g_ref[e, pl.ds(L, L)] = bb
                    return 0

                lax.fori_loop(0, SUB // L, grp, 0)

        def copy_scatter_idx(b):
            for j in range(KSUB):
                def cp(g, _, j=j):
                    rsc[b, j, pl.ds(g * L, L)] = rowv[b, pl.ds(j * SUB + g * L, L)]
                    return 0

                lax.fori_loop(0, SUB // L, cp, 0)

        def fire_scatter(b):
            for j in range(KSUB):
                pltpu.async_copy(gbufs[b].at[pl.ds(j * SUB, SUB)],
                                 accum.at[rsc.at[b, j]], sss[b], add=True)

        def drain_scatter(b):
            for j in range(KSUB):
                pltpu.make_async_copy(gbufs[b].at[pl.ds(j * SUB, SUB)],
                                      accum.at[pl.ds(0, SUB)], sss[b]).wait()

        def step(c, b, first, last):
            drain_gathers(b)
            drain_loads(1 - b)
            if not first:
                drain_scatter(1 - b)
            fire_gathers(1 - b)
            copy_scatter_idx(b)
            scale(b)
            fire_scatter(b)
            r_next = jnp.minimum(c + 2, NCH - 1)
            fire_loads(r_next, b)

        fire_loads(0, 0)
        drain_loads(0)
        fire_gathers(0)
        fire_loads(1, 1)
        step(0, 0, first=True, last=False)
        step(1, 1, first=False, last=False)

        def pair(p, _):
            for b in (0, 1):
                step(2 * p + 2 + b, b, first=False, last=False)
            return 0

        lax.fori_loop(0, (NCH - 3) // 2, pair, 0)

        drain_gathers(0)
        drain_scatter(1)
        copy_scatter_idx(0)
        scale(0)
        fire_scatter(0)
        drain_scatter(0)
        drain_loads(1)
        plsc.subcore_barrier()

        nq = (NPAD // NS) // ZROWS

        def rd_in(q, b):
            r0 = sid * (NPAD // NS) + q * ZROWS
            pltpu.async_copy(accum.at[pl.ds(r0, ZROWS)],
                             gbufs[b].at[pl.ds(0, ZROWS)], sgs[b])

        def rd_in_wait(b):
            pltpu.make_async_copy(emb_h.at[pl.ds(0, ZROWS)],
                                  gbufs[b].at[pl.ds(0, ZROWS)], sgs[b]).wait()

        def rd_out(q, b):
            r0 = sid * (NPAD // NS) + q * ZROWS

            @pl.when(cid == 0)
            def _():
                pltpu.async_copy(gbufs[b].at[pl.ds(0, ZROWS)],
                                 out0.at[pl.ds(r0, ZROWS)], sss[b])

            @pl.when(cid == 1)
            def _():
                pltpu.async_copy(gbufs[b].at[pl.ds(0, ZROWS)],
                                 out1.at[pl.ds(r0, ZROWS)], sss[b])

        def rd_out_wait(b):
            pltpu.make_async_copy(gbufs[b].at[pl.ds(0, ZROWS)],
                                  out0.at[pl.ds(0, ZROWS)], sss[b]).wait()

        rd_in(0, 0)
        for q in range(1, nq + 1):
            b = q % 2
            if q < nq:
                if q >= 2:
                    rd_out_wait(b)
                rd_in(q, b)
            rd_in_wait(1 - b)
            rd_out(q - 1, 1 - b)
        rd_out_wait(0)
        rd_out_wait(1)

    return k(adj_row, adj_col, adj_val, emb)


def _sc_gather3(t0, t1, t2, idx, nrows):
    mesh = plsc.VectorSubcoreMesh(core_axis_name="c", subcore_axis_name="s")
    per_w = nrows // NW
    kq = per_w // 128

    @functools.partial(
        pl.kernel,
        mesh=mesh,
        out_type=jax.ShapeDtypeStruct((nrows, D), jnp.float32),
        scratch_types=[
            pltpu.VMEM((per_w,), jnp.int32),
            pltpu.VMEM((per_w, D), jnp.float32),
            pltpu.VMEM((per_w, D), jnp.float32),
            pltpu.VMEM((per_w, D), jnp.float32),
            pltpu.SemaphoreType.DMA,
        ],
        compiler_params=pltpu.CompilerParams(use_tc_tiling_on_sc=False),
    )
    def k(t0_h, t1_h, t2_h, idx_h, out_h, idxv, b0, b1, b2, sem):
        wid = lax.axis_index("c") * NS + lax.axis_index("s")
        pltpu.sync_copy(idx_h.at[pl.ds(wid * per_w, per_w)], idxv)
        copies = []
        for tab, buf in ((t0_h, b0), (t1_h, b1), (t2_h, b2)):
            for q in range(kq):
                copies.append(
                    pltpu.async_copy(tab.at[idxv.at[pl.ds(q * 128, 128)]],
                                     buf.at[pl.ds(q * 128, 128)], sem))
        for cpy in copies:
            cpy.wait()

        def addrow(r, _):
            a0 = b0[r, pl.ds(0, L)] + b1[r, pl.ds(0, L)] + b2[r, pl.ds(0, L)]
            a1 = b0[r, pl.ds(L, L)] + b1[r, pl.ds(L, L)] + b2[r, pl.ds(L, L)]
            b0[r, pl.ds(0, L)] = a0
            b0[r, pl.ds(L, L)] = a1
            return 0

        lax.fori_loop(0, per_w, addrow, 0)
        pltpu.sync_copy(b0, out_h.at[pl.ds(wid * per_w, per_w)])

    return k(t0, t1, t2, idx)


def _sc_add2(a, b):
    mesh = plsc.VectorSubcoreMesh(core_axis_name="c", subcore_axis_name="s")
    RPW = NPAD // NW
    RC = 391

    @functools.partial(
        pl.kernel,
        mesh=mesh,
        out_type=jax.ShapeDtypeStruct((NPAD, D), jnp.float32),
        scratch_types=[
            pltpu.VMEM((RC, D), jnp.float32),
            pltpu.VMEM((RC, D), jnp.float32),
            pltpu.SemaphoreType.DMA,
        ],
        compiler_params=pltpu.CompilerParams(use_tc_tiling_on_sc=False),
    )
    def k(a_h, b_h, o_h, abuf, bbuf, sem):
        wid = lax.axis_index("c") * NS + lax.axis_index("s")

        def chunk(q, _):
            r0 = wid * RPW + q * RC
            ca = pltpu.async_copy(a_h.at[pl.ds(r0, RC)], abuf, sem)
            cb = pltpu.async_copy(b_h.at[pl.ds(r0, RC)], bbuf, sem)
            ca.wait()
            cb.wait()

            def addrow(r, _):
                abuf[r, pl.ds(0, L)] = abuf[r, pl.ds(0, L)] + bbuf[r, pl.ds(0, L)]
                abuf[r, pl.ds(L, L)] = abuf[r, pl.ds(L, L)] + bbuf[r, pl.ds(L, L)]
                return 0

            lax.fori_loop(0, RC, addrow, 0)
            pltpu.sync_copy(abuf, o_h.at[pl.ds(r0, RC)])
            return 0

        lax.fori_loop(0, RPW // RC, chunk, 0)

    return k(a, b)


def _sc_concat_tables(user_table, item_table):
    mesh = plsc.VectorSubcoreMesh(core_axis_name="c", subcore_axis_name="s")
    UPW = N_USERS // NS
    IPW = N_ITEMS // NS
    RC = 625

    @functools.partial(
        pl.kernel,
        mesh=mesh,
        out_type=jax.ShapeDtypeStruct((N, D), jnp.float32),
        scratch_types=[
            pltpu.VMEM((RC, D), jnp.float32),
            pltpu.SemaphoreType.DMA,
        ],
        compiler_params=pltpu.CompilerParams(use_tc_tiling_on_sc=False),
    )
    def k(u_h, i_h, o_h, buf, sem):
        cid = lax.axis_index("c")
        sid = lax.axis_index("s")

        @pl.when(cid == 0)
        def _():
            def chunk(q, _):
                r0 = sid * UPW + q * RC
                pltpu.async_copy(u_h.at[pl.ds(r0, RC)], buf, sem).wait()
                pltpu.sync_copy(buf, o_h.at[pl.ds(r0, RC)])
                return 0

            lax.fori_loop(0, UPW // RC, chunk, 0)

        @pl.when(cid == 1)
        def _():
            def chunk(q, _):
                r0 = sid * IPW + q * RC
                pltpu.async_copy(i_h.at[pl.ds(r0, RC)], buf, sem).wait()
                pltpu.sync_copy(buf, o_h.at[pl.ds(N_USERS + r0, RC)])
                return 0

            lax.fori_loop(0, IPW // RC, chunk, 0)

    return k(user_table, item_table)


def _tc_loss(rows, wu_c, wp_c):
    RB = 1024

    def body(rows_ref, wuc_ref, wpc_ref, o_ref, un_ref, pn_ref):
        def norm(x):
            return x / (jnp.sqrt(jnp.sum(x * x, axis=1, keepdims=True)) + 1e-12)

        ue = norm(rows_ref[pl.ds(0, B), :])
        ie = norm(rows_ref[pl.ds(B, B), :])
        un_ref[...] = norm(rows_ref[pl.ds(2 * B, B), :]).astype(jnp.bfloat16)
        pn_ref[...] = norm(rows_ref[pl.ds(3 * B, B), :]).astype(jnp.bfloat16)

        diff = ue - ie
        d = jnp.sqrt(jnp.sum(diff * diff, axis=1))
        t = d + 1e-12
        align = jnp.sum(t * t) / B

        def uniform(xn_ref, wc_ref):
            w_full = wc_ref[...]

            def blkstep(k, s):
                xb = xn_ref[pl.ds(k * RB, RB), :]
                g = lax.dot_general(xb, xn_ref[...],
                                    (((1,), (1,)), ((), ())),
                                    preferred_element_type=jnp.float32)
                sq = jnp.maximum(2.0 - 2.0 * g, 0.0)
                e = jnp.exp(-T_CONST * sq)
                ew = lax.dot_general(e, w_full, (((1,), (0,)), ((), ())),
                                     preferred_element_type=jnp.float32)
                wc = wc_ref[pl.ds(k * RB, RB), :]
                return s + jnp.sum(ew * wc)

            s = lax.fori_loop(0, B // RB, blkstep, 0.0)
            n = jnp.sum(w_full)
            return jnp.log((s - n) / (n * (n - 1.0)) + 1e-12)

        lu = uniform(un_ref, wuc_ref)
        lp = uniform(pn_ref, wpc_ref)
        uni = GAMMA * (lu + lp) / 2.0

        li = lax.broadcasted_iota(jnp.int32, (1, 128), 1)
        o_ref[...] = jnp.where(li == 0, align,
                               jnp.where(li == 1, uni, 0.0))

    return pl.pallas_call(
        body,
        out_shape=jax.ShapeDtypeStruct((1, 128), jnp.float32),
        scratch_shapes=[
            pltpu.VMEM((B, D), jnp.bfloat16),
            pltpu.VMEM((B, D), jnp.bfloat16),
        ],
    )(rows, wu_c, wp_c)


def kernel(user, positive, adj_row, adj_col, adj_val, user_table, item_table):
    user = user.astype(jnp.int32)
    positive = positive.astype(jnp.int32)
    adj_row = adj_row.astype(jnp.int32)
    adj_col = adj_col.astype(jnp.int32)

    emb0 = _sc_concat_tables(user_table, item_table)

    p1a, p1b = _spmm_kernel(adj_row, adj_col, adj_val, emb0)
    emb1 = _sc_add2(p1a, p1b)
    p2a, p2b = _spmm_kernel(adj_row, adj_col, adj_val, emb1)

    su = jnp.sort(user)
    sp = jnp.sort(positive)
    cat_idx = jnp.concatenate([user, N_USERS + positive, su, N_USERS + sp])
    rows = _sc_gather3(emb1, p2a, p2b, cat_idx, 4 * B)

    wu = jnp.concatenate(
        [jnp.ones((1,), jnp.float32), (su[1:] != su[:-1]).astype(jnp.float32)])
    wp = jnp.concatenate(
        [jnp.ones((1,), jnp.float32), (sp[1:] != sp[:-1]).astype(jnp.float32)])
    o = _tc_loss(rows, wu.reshape(B, 1), wp.reshape(B, 1))
    return jnp.stack([o[0, 0], o[0, 1]])
